# baseline (speedup 1.0000x reference)
import functools

import jax
import jax.numpy as jnp
from jax import lax
from jax.experimental import pallas as pl
from jax.experimental.pallas import tpu as pltpu
from jax.experimental.pallas import tpu_sc as plsc

DIM_ = 4096
BM_ = 512


def _sc_gather_body(idx_hbm, gamma_hbm, beta_hbm, g_out, b_out, idx_v, row_v, sem):
    wid = lax.axis_index("s") * 2 + lax.axis_index("c")

    @pl.when(wid == 0)
    def _():
        pltpu.sync_copy(idx_hbm, idx_v)
        pltpu.async_copy(gamma_hbm.at[idx_v], row_v, sem).wait()
        pltpu.sync_copy(row_v, g_out)

    @pl.when(wid == 1)
    def _():
        pltpu.sync_copy(idx_hbm, idx_v)
        pltpu.async_copy(beta_hbm.at[idx_v], row_v, sem).wait()
        pltpu.sync_copy(row_v, b_out)


def _sc_gather(idx, gamma, beta):
    D = gamma.shape[1]
    mesh = plsc.VectorSubcoreMesh(core_axis_name="c", subcore_axis_name="s")
    f = functools.partial(
        pl.kernel,
        out_type=[
            jax.ShapeDtypeStruct((1, D), jnp.float32),
            jax.ShapeDtypeStruct((1, D), jnp.float32),
        ],
        mesh=mesh,
        scratch_types=[
            pltpu.VMEM((1,), jnp.int32),
            pltpu.VMEM((1, D), jnp.float32),
            pltpu.SemaphoreType.DMA,
        ],
    )(_sc_gather_body)
    return f(idx, gamma, beta)


def _affine_body(feat_ref, g_ref, b_ref, out_ref):
    out_ref[...] = feat_ref[...] * g_ref[...] + b_ref[...]


def kernel(feat, modality_id, gamma, beta):
    B, D = feat.shape
    idx = jnp.asarray(modality_id, jnp.int32).reshape(1)
    g_row, b_row = _sc_gather(idx, gamma, beta)
    grid = (B // BM_,)
    return pl.pallas_call(
        _affine_body,
        grid=grid,
        in_specs=[
            pl.BlockSpec((BM_, D), lambda i: (i, 0)),
            pl.BlockSpec((1, D), lambda i: (0, 0)),
            pl.BlockSpec((1, D), lambda i: (0, 0)),
        ],
        out_specs=pl.BlockSpec((BM_, D), lambda i: (i, 0)),
        out_shape=jax.ShapeDtypeStruct((B, D), feat.dtype),
        compiler_params=pltpu.CompilerParams(
            dimension_semantics=("arbitrary",),
        ),
    )(feat, g_row, b_row)


# speedup vs baseline: 1.0006x; 1.0006x over previous
import functools

import jax
import jax.numpy as jnp
from jax import lax
from jax.experimental import pallas as pl
from jax.experimental.pallas import tpu as pltpu
from jax.experimental.pallas import tpu_sc as plsc

DIM_ = 4096
BM_ = 512


def _scs_gather_body(idx_hbm, gamma_hbm, beta_hbm, g_out, b_out, idx_s):
    cid = lax.axis_index("c")

    @pl.when(cid == 0)
    def _():
        pltpu.sync_copy(idx_hbm, idx_s)
        i = idx_s[0]
        pltpu.sync_copy(gamma_hbm.at[pl.ds(i, 1)], g_out)
        pltpu.sync_copy(beta_hbm.at[pl.ds(i, 1)], b_out)


def _sc_gather(idx, gamma, beta):
    D = gamma.shape[1]
    mesh = plsc.ScalarSubcoreMesh(axis_name="c", num_cores=2)
    f = functools.partial(
        pl.kernel,
        out_type=[
            jax.ShapeDtypeStruct((1, D), jnp.float32),
            jax.ShapeDtypeStruct((1, D), jnp.float32),
        ],
        mesh=mesh,
        scratch_types=[
            pltpu.SMEM((1,), jnp.int32),
        ],
    )(_scs_gather_body)
    return f(idx, gamma, beta)


def _affine_body(feat_ref, g_ref, b_ref, out_ref):
    out_ref[...] = feat_ref[...] * g_ref[...] + b_ref[...]


def kernel(feat, modality_id, gamma, beta):
    B, D = feat.shape
    idx = jnp.asarray(modality_id, jnp.int32).reshape(1)
    g_row, b_row = _sc_gather(idx, gamma, beta)
    grid = (B // BM_,)
    return pl.pallas_call(
        _affine_body,
        grid=grid,
        in_specs=[
            pl.BlockSpec((BM_, D), lambda i: (i, 0)),
            pl.BlockSpec((1, D), lambda i: (0, 0)),
            pl.BlockSpec((1, D), lambda i: (0, 0)),
        ],
        out_specs=pl.BlockSpec((BM_, D), lambda i: (i, 0)),
        out_shape=jax.ShapeDtypeStruct((B, D), feat.dtype),
        compiler_params=pltpu.CompilerParams(
            dimension_semantics=("arbitrary",),
        ),
    )(feat, g_row, b_row)
